# baseline (speedup 1.0000x reference)
import functools

import jax
import jax.numpy as jnp
from jax import lax
from jax.experimental import pallas as pl
from jax.experimental.pallas import tpu as pltpu
from jax.experimental.pallas import tpu_sc as plsc


def _make_gather(b: int, l: int, vocab: int, d: int):
    info = plsc.get_sparse_core_info()
    nc, ns = info.num_cores, info.num_subcores
    nw = nc * ns
    lanes = info.num_lanes

    bt_n, bl_n = nw, b // nw
    ct_n, cs_n = d // 8, 8
    lt_n, ls_n = l // 8, 8
    chunk = ls_n * bl_n
    pitch = bl_n + 1
    assert bl_n == 128 and d == 32 and l % 8 == 0

    mesh = plsc.VectorSubcoreMesh(core_axis_name="c", subcore_axis_name="s")

    @functools.partial(
        pl.kernel,
        mesh=mesh,
        compiler_params=pltpu.CompilerParams(
            use_tc_tiling_on_sc=False, needs_layout_passes=False
        ),
        out_type=jax.ShapeDtypeStruct((l, ct_n, bt_n, cs_n, bl_n), jnp.float32),
        scratch_types=[
            pltpu.VMEM((bl_n, l), jnp.int32),
            pltpu.VMEM((chunk,), jnp.int32),
            pltpu.VMEM((chunk,), jnp.int32),
            pltpu.VMEM((chunk, d), jnp.float32),
            pltpu.VMEM((chunk, d), jnp.float32),
            pltpu.VMEM((ls_n, ct_n, cs_n, pitch), jnp.float32),
            pltpu.SemaphoreType.DMA,
            pltpu.SemaphoreType.DMA,
            pltpu.SemaphoreType.DMA,
        ],
    )
    def gather_kernel(
        idx_hbm, table_hbm, z_hbm, idx_vm, if0, if1, rv0, rv1, zbuf, sg0, sg1, so
    ):
        w = lax.axis_index("s") * nc + lax.axis_index("c")
        idx_f = (if0, if1)
        rows = (rv0, rv1)
        sem_g = (sg0, sg1)
        pltpu.sync_copy(idx_hbm.at[pl.ds(w * bl_n, bl_n), :], idx_vm)
        lane = lax.iota(jnp.int32, lanes)
        ct_v = [jnp.right_shift(16 * h + lane, 3) for h in range(2)]
        cs_v = [jnp.bitwise_and(16 * h + lane, 7) for h in range(2)]

        def build(lt, p):
            def one(k8, c2):
                for j in range(8):
                    k = k8 * 8 + j
                    e = k * lanes + lane
                    vals = plsc.load_gather(
                        idx_vm,
                        [jnp.right_shift(e, 3), 8 * lt + jnp.bitwise_and(e, 7)],
                    )
                    idx_f[p][pl.ds(k * lanes, lanes)] = vals
                return c2

            lax.fori_loop(0, chunk // lanes // 8, one, 0)

        def gather_copy(p):
            return pltpu.make_async_copy(table_hbm.at[idx_f[p]], rows[p], sem_g[p])

        def out_copy(lt):
            return pltpu.make_async_copy(
                zbuf.at[:, :, :, pl.ds(0, bl_n)],
                z_hbm.at[pl.ds(lt * ls_n, ls_n), :, w],
                so,
            )

        def shuffle(p):
            def grp_body(g, c2):
                ls = jnp.right_shift(g, 2)
                blg = jnp.bitwise_and(g, 3)
                ls_s = jnp.broadcast_to(ls, (lanes,))
                for bl_i in range(2 * lanes):
                    bl = blg * 2 * lanes + bl_i
                    r = 8 * bl + ls
                    bl_s = jnp.broadcast_to(bl, (lanes,))
                    for h in range(2):
                        vals = rows[p][r, pl.ds(h * lanes, lanes)]
                        plsc.store_scatter(
                            zbuf, [ls_s, ct_v[h], cs_v[h], bl_s], vals
                        )
                return c2

            lax.fori_loop(0, ls_n * 4, grp_body, 0)

        build(0, 0)
        gather_copy(0).start()

        def pair_body(g, carry):
            for par in range(2):
                i = 2 * g + par
                gather_copy(par).wait()
                build(i + 1, 1 - par)
                gather_copy(1 - par).start()
                lax.cond(i > 0, lambda: out_copy(i - 1).wait(), lambda: None)
                shuffle(par)
                out_copy(i).start()
            return carry

        lax.fori_loop(0, (lt_n - 1) // 2, pair_body, 0)

        last = lt_n - 1
        gather_copy(last % 2).wait()
        out_copy(last - 1).wait()
        shuffle(last % 2)
        out_copy(last).start()
        out_copy(last).wait()

    return gather_kernel


def kernel(indices, table):
    b, l = indices.shape
    vocab, d = table.shape
    z = _make_gather(b, l, vocab, d)(indices, table)
    return z.transpose((2, 4, 0, 1, 3)).reshape(b, l, d)


# speedup vs baseline: 1.0057x; 1.0057x over previous
import functools

import jax
import jax.numpy as jnp
from jax import lax
from jax.experimental import pallas as pl
from jax.experimental.pallas import tpu as pltpu
from jax.experimental.pallas import tpu_sc as plsc


def _make_gather(b: int, l: int, vocab: int, d: int):
    info = plsc.get_sparse_core_info()
    nc, ns = info.num_cores, info.num_subcores
    nw = nc * ns
    lanes = info.num_lanes

    bt_n, bl_n = nw, b // nw
    ct_n, cs_n = d // 8, 8
    lt_n, ls_n = l // 8, 8
    chunk = ls_n * bl_n
    pitch = bl_n + 1
    assert bl_n == 128 and d == 32 and l % 8 == 0

    mesh = plsc.VectorSubcoreMesh(core_axis_name="c", subcore_axis_name="s")

    @functools.partial(
        pl.kernel,
        mesh=mesh,
        compiler_params=pltpu.CompilerParams(
            use_tc_tiling_on_sc=False, needs_layout_passes=False
        ),
        out_type=jax.ShapeDtypeStruct((l, ct_n, bt_n, cs_n, bl_n), jnp.float32),
        scratch_types=[
            pltpu.VMEM((bl_n, l), jnp.int32),
            pltpu.VMEM((chunk,), jnp.int32),
            pltpu.VMEM((chunk,), jnp.int32),
            pltpu.VMEM((chunk, d), jnp.float32),
            pltpu.VMEM((chunk, d), jnp.float32),
            pltpu.VMEM((ls_n, ct_n, cs_n, pitch), jnp.float32),
            pltpu.SemaphoreType.DMA,
            pltpu.SemaphoreType.DMA,
            pltpu.SemaphoreType.DMA,
        ],
    )
    def gather_kernel(
        idx_hbm, table_hbm, z_hbm, idx_vm, if0, if1, rv0, rv1, zbuf, sg0, sg1, so
    ):
        w = lax.axis_index("s") * nc + lax.axis_index("c")
        idx_f = (if0, if1)
        rows = (rv0, rv1)
        sem_g = (sg0, sg1)
        pltpu.sync_copy(idx_hbm.at[pl.ds(w * bl_n, bl_n), :], idx_vm)
        lane = lax.iota(jnp.int32, lanes)
        ct_v = [jnp.right_shift(16 * h + lane, 3) for h in range(2)]
        cs_v = [jnp.bitwise_and(16 * h + lane, 7) for h in range(2)]

        def build(lt, p):
            def one(k, c2):
                e = k * lanes + lane
                vals = plsc.load_gather(
                    idx_vm,
                    [jnp.right_shift(e, 3), 8 * lt + jnp.bitwise_and(e, 7)],
                )
                idx_f[p][pl.ds(k * lanes, lanes)] = vals
                return c2

            lax.fori_loop(0, chunk // lanes, one, 0)

        def gather_copy(p):
            return pltpu.make_async_copy(table_hbm.at[idx_f[p]], rows[p], sem_g[p])

        def out_copy(lt):
            return pltpu.make_async_copy(
                zbuf.at[:, :, :, pl.ds(0, bl_n)],
                z_hbm.at[pl.ds(lt * ls_n, ls_n), :, w],
                so,
            )

        def shuffle(p):
            def grp_body(g, c2):
                ls = jnp.right_shift(g, 3)
                blg = jnp.bitwise_and(g, 7)
                ls_s = jnp.broadcast_to(ls, (lanes,))
                for bl_i in range(lanes):
                    bl = blg * lanes + bl_i
                    r = 8 * bl + ls
                    bl_s = jnp.broadcast_to(bl, (lanes,))
                    for h in range(2):
                        vals = rows[p][r, pl.ds(h * lanes, lanes)]
                        plsc.store_scatter(
                            zbuf, [ls_s, ct_v[h], cs_v[h], bl_s], vals
                        )
                return c2

            lax.fori_loop(0, ls_n * 8, grp_body, 0)

        build(0, 0)
        gather_copy(0).start()

        def pair_body(g, carry):
            for par in range(2):
                i = 2 * g + par
                gather_copy(par).wait()
                build(i + 1, 1 - par)
                gather_copy(1 - par).start()
                lax.cond(i > 0, lambda: out_copy(i - 1).wait(), lambda: None)
                shuffle(par)
                out_copy(i).start()
            return carry

        lax.fori_loop(0, (lt_n - 1) // 2, pair_body, 0)

        last = lt_n - 1
        gather_copy(last % 2).wait()
        out_copy(last - 1).wait()
        shuffle(last % 2)
        out_copy(last).start()
        out_copy(last).wait()

    return gather_kernel


def kernel(indices, table):
    b, l = indices.shape
    vocab, d = table.shape
    z = _make_gather(b, l, vocab, d)(indices, table)
    return z.transpose((2, 4, 0, 1, 3)).reshape(b, l, d)
